# baseline (speedup 1.0000x reference)
import functools

import jax
import jax.numpy as jnp
from jax import lax
from jax.experimental import pallas as pl
from jax.experimental.pallas import tpu as pltpu
from jax.experimental.pallas import tpu_sc as plsc

N = 10000
D = 128
E = 320000

NTILES = 16
NPAD = 10240
ROWS_PER_TILE = NPAD // NTILES
ZBLK = 128
CHUNK = 128
CHUNKS_PER_TILE = -(-E // (NTILES * CHUNK))
EPT = CHUNK * CHUNKS_PER_TILE
EPAD = EPT * NTILES
LANES = 16


def _mm_body(x_ref, wt_ref, b_ref, o_ref):
    o_ref[...] = (
        jnp.dot(x_ref[...], wt_ref[...],
                preferred_element_type=jnp.float32,
                precision=lax.Precision.HIGHEST)
        + b_ref[...]
    )


def _linear(x, Wt, b):
    M = x.shape[0]
    BM = 1000
    return pl.pallas_call(
        _mm_body,
        grid=(M // BM,),
        in_specs=[
            pl.BlockSpec((BM, D), lambda i: (i, 0)),
            pl.BlockSpec((D, D), lambda i: (0, 0)),
            pl.BlockSpec((1, D), lambda i: (0, 0)),
        ],
        out_specs=pl.BlockSpec((BM, D), lambda i: (i, 0)),
        out_shape=jax.ShapeDtypeStruct((M, D), jnp.float32),
    )(x, Wt, b)


def _div_body(sa_ref, ca_ref, sb_ref, cb_ref, oa_ref, ob_ref):
    oa_ref[...] = sa_ref[...] / jnp.maximum(ca_ref[...], 1.0)
    ob_ref[...] = sb_ref[...] / jnp.maximum(cb_ref[...], 1.0)


def _mean(sum_a, cnt_a, sum_b, cnt_b):
    BM = 1024
    grid = NPAD // BM
    return pl.pallas_call(
        _div_body,
        grid=(grid,),
        in_specs=[
            pl.BlockSpec((BM, D), lambda i: (i, 0)),
            pl.BlockSpec((BM, 1), lambda i: (i, 0)),
            pl.BlockSpec((BM, D), lambda i: (i, 0)),
            pl.BlockSpec((BM, 1), lambda i: (i, 0)),
        ],
        out_specs=[
            pl.BlockSpec((BM, D), lambda i: (i, 0)),
            pl.BlockSpec((BM, D), lambda i: (i, 0)),
        ],
        out_shape=[
            jax.ShapeDtypeStruct((NPAD, D), jnp.float32),
            jax.ShapeDtypeStruct((NPAD, D), jnp.float32),
        ],
    )(sum_a, cnt_a.reshape(NPAD, 1), sum_b, cnt_b.reshape(NPAD, 1))


@functools.partial(
    pl.kernel,
    out_type=[
        jax.ShapeDtypeStruct((NPAD, D), jnp.float32),
        jax.ShapeDtypeStruct((NPAD,), jnp.float32),
        jax.ShapeDtypeStruct((NPAD, D), jnp.float32),
        jax.ShapeDtypeStruct((NPAD,), jnp.float32),
    ],
    mesh=plsc.VectorSubcoreMesh(core_axis_name="c", subcore_axis_name="s"),
    scratch_types=[
        pltpu.VMEM((ZBLK, D), jnp.float32),
        pltpu.VMEM((ZBLK, D), jnp.float32),
        pltpu.VMEM((ROWS_PER_TILE,), jnp.float32),
        pltpu.VMEM((CHUNK,), jnp.float32),
        pltpu.VMEM((CHUNK,), jnp.int32),
        pltpu.VMEM((CHUNK,), jnp.int32),
        pltpu.VMEM((CHUNK,), jnp.int32),
        pltpu.VMEM((CHUNK,), jnp.int32),
        pltpu.VMEM_SHARED((NPAD, D), jnp.float32),
        pltpu.VMEM_SHARED((NPAD,), jnp.float32),
        pltpu.SemaphoreType.DMA,
        pltpu.SemaphoreType.DMA,
        pltpu.SemaphoreType.DMA,
        pltpu.SemaphoreType.DMA,
        pltpu.SemaphoreType.DMA,
    ],
)
def _sc_aggregate(wh_attr, src_of, dst_of, wh_vul, src_has, dst_has,
                  sum_vul, cnt_vul, sum_attr, cnt_attr,
                  gbuf, gbuf1, cbuf, ones_v, src_v, src_v1,
                  dst_v, dst_v1, acc, cnt, sem, sem1, isem, ssem, ssem1):
    c = lax.axis_index("c")
    s = lax.axis_index("s")

    def etype(table, srcs, dsts, sum_out, cnt_out):
        row0 = s * ROWS_PER_TILE

        def zfill(r, carry):
            for j in range(D // LANES):
                gbuf[r, pl.ds(LANES * j, LANES)] = jnp.zeros((LANES,), jnp.float32)
            return carry

        lax.fori_loop(0, ZBLK, zfill, 0)

        def cfill(r, carry):
            cbuf[pl.ds(r * LANES, LANES)] = jnp.zeros((LANES,), jnp.float32)
            return carry

        lax.fori_loop(0, ROWS_PER_TILE // LANES, cfill, 0)

        def ofill(r, carry):
            ones_v[pl.ds(r * LANES, LANES)] = jnp.ones((LANES,), jnp.float32)
            return carry

        lax.fori_loop(0, CHUNK // LANES, ofill, 0)

        for m in range(ROWS_PER_TILE // ZBLK):
            pltpu.sync_copy(gbuf, acc.at[pl.ds(row0 + m * ZBLK, ZBLK)])
        pltpu.sync_copy(cbuf, cnt.at[pl.ds(row0, ROWS_PER_TILE)])
        plsc.subcore_barrier()

        ebase = s * EPT

        def idx_off(j):
            return ebase + j * CHUNK

        def load_idx_async(j, sv, dv):
            pltpu.async_copy(srcs.at[pl.ds(idx_off(j), CHUNK)], sv, isem)
            pltpu.async_copy(dsts.at[pl.ds(idx_off(j), CHUNK)], dv, isem)

        def wait_idx(j, sv, dv):
            pltpu.make_async_copy(srcs.at[pl.ds(idx_off(j), CHUNK)], sv, isem).wait()
            pltpu.make_async_copy(dsts.at[pl.ds(idx_off(j), CHUNK)], dv, isem).wait()

        NCH = CHUNKS_PER_TILE

        def drain_scatter(gb, dv, psem):
            pltpu.make_async_copy(gb, acc.at[dv], psem).wait()

        pltpu.sync_copy(srcs.at[pl.ds(idx_off(0), CHUNK)], src_v)
        pltpu.sync_copy(dsts.at[pl.ds(idx_off(0), CHUNK)], dst_v)
        pltpu.async_copy(table.at[src_v], gbuf, sem)
        load_idx_async(1, src_v1, dst_v1)
        pltpu.make_async_copy(table.at[src_v], gbuf, sem).wait()
        pltpu.async_copy(gbuf, acc.at[dst_v], ssem, add=True)
        pltpu.sync_copy(ones_v, cnt.at[dst_v], add=True)
        wait_idx(1, src_v1, dst_v1)
        pltpu.async_copy(table.at[src_v1], gbuf1, sem1)

        def pair(i, carry):
            j1 = 2 * i + 1
            drain_scatter(gbuf, dst_v, ssem)
            load_idx_async(j1 + 1, src_v, dst_v)
            pltpu.make_async_copy(table.at[src_v1], gbuf1, sem1).wait()
            pltpu.async_copy(gbuf1, acc.at[dst_v1], ssem1, add=True)
            pltpu.sync_copy(ones_v, cnt.at[dst_v1], add=True)
            wait_idx(j1 + 1, src_v, dst_v)
            pltpu.async_copy(table.at[src_v], gbuf, sem)
            j2 = j1 + 1
            drain_scatter(gbuf1, dst_v1, ssem1)

            @pl.when(j2 + 1 < NCH)
            def _():
                load_idx_async(j2 + 1, src_v1, dst_v1)

            pltpu.make_async_copy(table.at[src_v], gbuf, sem).wait()
            pltpu.async_copy(gbuf, acc.at[dst_v], ssem, add=True)
            pltpu.sync_copy(ones_v, cnt.at[dst_v], add=True)

            @pl.when(j2 + 1 < NCH)
            def _():
                wait_idx(j2 + 1, src_v1, dst_v1)
                pltpu.async_copy(table.at[src_v1], gbuf1, sem1)

            return carry

        lax.fori_loop(0, (NCH - 1) // 2, pair, 0)
        drain_scatter(gbuf, dst_v, ssem)
        plsc.subcore_barrier()

        for m in range(ROWS_PER_TILE // ZBLK):
            r0 = row0 + m * ZBLK
            pltpu.sync_copy(acc.at[pl.ds(r0, ZBLK)], gbuf)
            pltpu.sync_copy(gbuf, sum_out.at[pl.ds(r0, ZBLK)])
        pltpu.sync_copy(cnt.at[pl.ds(row0, ROWS_PER_TILE)], cbuf)
        pltpu.sync_copy(cbuf, cnt_out.at[pl.ds(row0, ROWS_PER_TILE)])

    @pl.when(c == 0)
    def _():
        etype(wh_attr, src_of, dst_of, sum_vul, cnt_vul)

    @pl.when(c == 1)
    def _():
        etype(wh_vul, src_has, dst_has, sum_attr, cnt_attr)


def kernel(x_vul, x_attr, edge_index_of, edge_index_has, W_of, b_of, W_has, b_has):
    wh_attr = _linear(x_attr, W_of.T, b_of.reshape(1, D))
    wh_vul = _linear(x_vul, W_has.T, b_has.reshape(1, D))

    pad = EPAD - E

    def padded(ei):
        src = jnp.concatenate([ei[0], jnp.zeros((pad,), jnp.int32)])
        dst = jnp.concatenate([ei[1], jnp.full((pad,), NPAD - 1, jnp.int32)])
        return src, dst

    so, do_ = padded(edge_index_of)
    sh, dh = padded(edge_index_has)

    sum_vul, cnt_vul, sum_attr, cnt_attr = _sc_aggregate(
        wh_attr, so, do_, wh_vul, sh, dh)
    h_vul, h_attr = _mean(sum_vul, cnt_vul, sum_attr, cnt_attr)
    return h_vul[:N], h_attr[:N]


# speedup vs baseline: 1.1350x; 1.1350x over previous
import functools

import jax
import jax.numpy as jnp
from jax import lax
from jax.experimental import pallas as pl
from jax.experimental.pallas import tpu as pltpu
from jax.experimental.pallas import tpu_sc as plsc

N = 10000
D = 128
E = 320000

NTILES = 16
NPAD = 10240
ROWS_PER_TILE = NPAD // NTILES
ZBLK = 128
CHUNK = 128
CHUNKS_PER_TILE = -(-E // (NTILES * CHUNK))
EPT = CHUNK * CHUNKS_PER_TILE
EPAD = EPT * NTILES
LANES = 16


def _mm_body(x_ref, wt_ref, b_ref, o_ref):
    o_ref[...] = (
        jnp.dot(x_ref[...], wt_ref[...],
                preferred_element_type=jnp.float32,
                precision=lax.Precision.HIGHEST)
        + b_ref[...]
    )


def _linear(x, Wt, b):
    M = x.shape[0]
    BM = 1000
    return pl.pallas_call(
        _mm_body,
        grid=(M // BM,),
        in_specs=[
            pl.BlockSpec((BM, D), lambda i: (i, 0)),
            pl.BlockSpec((D, D), lambda i: (0, 0)),
            pl.BlockSpec((1, D), lambda i: (0, 0)),
        ],
        out_specs=pl.BlockSpec((BM, D), lambda i: (i, 0)),
        out_shape=jax.ShapeDtypeStruct((M, D), jnp.float32),
    )(x, Wt, b)


def _div_body(sa_ref, ca_ref, sb_ref, cb_ref, oa_ref, ob_ref):
    oa_ref[...] = sa_ref[...] / jnp.maximum(ca_ref[...], 1.0)
    ob_ref[...] = sb_ref[...] / jnp.maximum(cb_ref[...], 1.0)


def _mean(sum_a, cnt_a, sum_b, cnt_b):
    BM = 1024
    grid = NPAD // BM
    return pl.pallas_call(
        _div_body,
        grid=(grid,),
        in_specs=[
            pl.BlockSpec((BM, D), lambda i: (i, 0)),
            pl.BlockSpec((BM, 1), lambda i: (i, 0)),
            pl.BlockSpec((BM, D), lambda i: (i, 0)),
            pl.BlockSpec((BM, 1), lambda i: (i, 0)),
        ],
        out_specs=[
            pl.BlockSpec((BM, D), lambda i: (i, 0)),
            pl.BlockSpec((BM, D), lambda i: (i, 0)),
        ],
        out_shape=[
            jax.ShapeDtypeStruct((NPAD, D), jnp.float32),
            jax.ShapeDtypeStruct((NPAD, D), jnp.float32),
        ],
    )(sum_a, cnt_a.reshape(NPAD, 1), sum_b, cnt_b.reshape(NPAD, 1))


@functools.partial(
    pl.kernel,
    out_type=[
        jax.ShapeDtypeStruct((NPAD, D), jnp.float32),
        jax.ShapeDtypeStruct((NPAD,), jnp.float32),
        jax.ShapeDtypeStruct((NPAD, D), jnp.float32),
        jax.ShapeDtypeStruct((NPAD,), jnp.float32),
    ],
    mesh=plsc.VectorSubcoreMesh(core_axis_name="c", subcore_axis_name="s"),
    scratch_types=[
        pltpu.VMEM((ZBLK, D), jnp.float32),
        pltpu.VMEM((ZBLK, D), jnp.float32),
        pltpu.VMEM((ROWS_PER_TILE,), jnp.float32),
        pltpu.VMEM((CHUNK,), jnp.float32),
        pltpu.VMEM((CHUNK,), jnp.int32),
        pltpu.VMEM((CHUNK,), jnp.int32),
        pltpu.VMEM((CHUNK,), jnp.int32),
        pltpu.VMEM((CHUNK,), jnp.int32),
        pltpu.VMEM((CHUNK,), jnp.int32),
        pltpu.VMEM_SHARED((NPAD, D), jnp.float32),
        pltpu.VMEM_SHARED((NPAD,), jnp.float32),
        pltpu.SemaphoreType.DMA,
        pltpu.SemaphoreType.DMA,
        pltpu.SemaphoreType.DMA,
        pltpu.SemaphoreType.DMA,
        pltpu.SemaphoreType.DMA,
    ],
)
def _sc_aggregate(wh_attr, src_of, dst_of, wh_vul, src_has, dst_has,
                  sum_vul, cnt_vul, sum_attr, cnt_attr,
                  gbuf, gbuf1, cbuf, ones_v, src_v, src_v1,
                  dst_v, dst_v1, dst_v2, acc, cnt, sem, sem1, isem, ssem, ssem1):
    c = lax.axis_index("c")
    s = lax.axis_index("s")

    def etype(table, srcs, dsts, sum_out, cnt_out):
        row0 = s * ROWS_PER_TILE

        def zfill(r, carry):
            for j in range(D // LANES):
                gbuf[r, pl.ds(LANES * j, LANES)] = jnp.zeros((LANES,), jnp.float32)
            return carry

        lax.fori_loop(0, ZBLK, zfill, 0)

        def cfill(r, carry):
            cbuf[pl.ds(r * LANES, LANES)] = jnp.zeros((LANES,), jnp.float32)
            return carry

        lax.fori_loop(0, ROWS_PER_TILE // LANES, cfill, 0)

        def ofill(r, carry):
            ones_v[pl.ds(r * LANES, LANES)] = jnp.ones((LANES,), jnp.float32)
            return carry

        lax.fori_loop(0, CHUNK // LANES, ofill, 0)

        for m in range(ROWS_PER_TILE // ZBLK):
            pltpu.sync_copy(gbuf, acc.at[pl.ds(row0 + m * ZBLK, ZBLK)])
        pltpu.sync_copy(cbuf, cnt.at[pl.ds(row0, ROWS_PER_TILE)])
        plsc.subcore_barrier()

        ebase = s * EPT

        def idx_off(j):
            return ebase + j * CHUNK

        def load_idx_async(j, sv, dv):
            pltpu.async_copy(srcs.at[pl.ds(idx_off(j), CHUNK)], sv, isem)
            pltpu.async_copy(dsts.at[pl.ds(idx_off(j), CHUNK)], dv, isem)

        def wait_idx(j, sv, dv):
            pltpu.make_async_copy(srcs.at[pl.ds(idx_off(j), CHUNK)], sv, isem).wait()
            pltpu.make_async_copy(dsts.at[pl.ds(idx_off(j), CHUNK)], dv, isem).wait()

        NCH = CHUNKS_PER_TILE

        GB = (gbuf, gbuf1)
        GS = (sem, sem1)
        SS = (ssem, ssem1)
        SV = (src_v, src_v1)
        DV = (dst_v, dst_v1, dst_v2)

        def chunk_body(j, q1, has_prev):
            p2, p3 = q1 % 2, q1 % 3
            n2, n3 = (q1 + 1) % 2, (q1 + 1) % 3
            m2, m3 = (q1 - 1) % 2, (q1 - 1) % 3
            if has_prev:
                pltpu.make_async_copy(GB[m2], acc.at[DV[m3]], SS[m2]).wait()

            @pl.when(j + 1 < NCH)
            def _():
                wait_idx(j + 1, SV[n2], DV[n3])
                pltpu.async_copy(table.at[SV[n2]], GB[n2], GS[n2])

            pltpu.make_async_copy(table.at[SV[p2]], GB[p2], GS[p2]).wait()
            pltpu.async_copy(GB[p2], acc.at[DV[p3]], SS[p2], add=True)
            pltpu.sync_copy(ones_v, cnt.at[DV[p3]], add=True)

            @pl.when(j + 2 < NCH)
            def _():
                load_idx_async(j + 2, SV[p2], DV[m3])

        pltpu.sync_copy(srcs.at[pl.ds(idx_off(0), CHUNK)], src_v)
        pltpu.sync_copy(dsts.at[pl.ds(idx_off(0), CHUNK)], dst_v)
        pltpu.async_copy(table.at[src_v], gbuf, sem)
        load_idx_async(1, src_v1, dst_v1)
        chunk_body(0, 0, has_prev=False)

        def six(i, carry):
            j0 = 6 * i + 1
            for q in range(6):
                chunk_body(j0 + q, (1 + q) % 6, has_prev=True)
            return carry

        lax.fori_loop(0, (NCH - 1) // 6, six, 0)
        pltpu.make_async_copy(GB[0], acc.at[DV[0]], SS[0]).wait()
        plsc.subcore_barrier()

        for m in range(ROWS_PER_TILE // ZBLK):
            r0 = row0 + m * ZBLK
            pltpu.sync_copy(acc.at[pl.ds(r0, ZBLK)], gbuf)
            pltpu.sync_copy(gbuf, sum_out.at[pl.ds(r0, ZBLK)])
        pltpu.sync_copy(cnt.at[pl.ds(row0, ROWS_PER_TILE)], cbuf)
        pltpu.sync_copy(cbuf, cnt_out.at[pl.ds(row0, ROWS_PER_TILE)])

    @pl.when(c == 0)
    def _():
        etype(wh_attr, src_of, dst_of, sum_vul, cnt_vul)

    @pl.when(c == 1)
    def _():
        etype(wh_vul, src_has, dst_has, sum_attr, cnt_attr)


def kernel(x_vul, x_attr, edge_index_of, edge_index_has, W_of, b_of, W_has, b_has):
    wh_attr = _linear(x_attr, W_of.T, b_of.reshape(1, D))
    wh_vul = _linear(x_vul, W_has.T, b_has.reshape(1, D))

    pad = EPAD - E

    def padded(ei):
        src = jnp.concatenate([ei[0], jnp.zeros((pad,), jnp.int32)])
        dst = jnp.concatenate([ei[1], jnp.full((pad,), NPAD - 1, jnp.int32)])
        return src, dst

    so, do_ = padded(edge_index_of)
    sh, dh = padded(edge_index_has)

    sum_vul, cnt_vul, sum_attr, cnt_attr = _sc_aggregate(
        wh_attr, so, do_, wh_vul, sh, dh)
    h_vul, h_attr = _mean(sum_vul, cnt_vul, sum_attr, cnt_attr)
    return h_vul[:N], h_attr[:N]
